# Initial kernel scaffold; baseline (speedup 1.0000x reference)
import jax
import jax.numpy as jnp
from jax.experimental import pallas as pl


def kernel(pert_batch, W, b):
    raise NotImplementedError("write your pallas kernel here")


# speedup vs baseline: 6.2142x; 6.2142x over previous
import functools

import jax
import jax.numpy as jnp
from jax import lax
from jax.experimental import pallas as pl
from jax.experimental.pallas import tpu as pltpu
from jax.experimental.pallas import tpu_sc as plsc

B, P, D, OUT = 4096, 32, 128, 128
NC, NS = 2, 16
NW = NC * NS
BPW = B // NW
BLK = 8
NBLK = BPW // BLK
LANES = 16
DC = D // LANES


def _seg_sum_body(x_hbm, out_hbm, buf, out_stage):
    c = lax.axis_index("c")
    s = lax.axis_index("s")
    wid = s * NC + c
    base = wid * BPW

    def gbody(g, carry):
        pltpu.sync_copy(x_hbm.at[pl.ds(base + g * BLK, BLK)], buf)
        for b in range(BLK):
            def pbody(p, accs):
                return tuple(accs[j] + buf[b, p, pl.ds(j * LANES, LANES)]
                             for j in range(DC))
            init = tuple(jnp.zeros((LANES,), jnp.float32) for _ in range(DC))
            accs = lax.fori_loop(0, P, pbody, init)
            for j in range(DC):
                out_stage[g * BLK + b, pl.ds(j * LANES, LANES)] = accs[j]
        return carry

    lax.fori_loop(0, NBLK, gbody, 0)
    pltpu.sync_copy(out_stage, out_hbm.at[pl.ds(base, BPW)])


_seg_sum = pl.kernel(
    _seg_sum_body,
    out_type=jax.ShapeDtypeStruct((B, D), jnp.float32),
    mesh=plsc.VectorSubcoreMesh(core_axis_name="c", subcore_axis_name="s"),
    scratch_types=[
        pltpu.VMEM((BLK, P, D), jnp.float32),
        pltpu.VMEM((BPW, D), jnp.float32),
    ],
)


def _mm_body(s_ref, w_ref, b_ref, o_ref):
    o_ref[...] = lax.dot_general(
        s_ref[...], w_ref[...],
        (((1,), (1,)), ((), ())),
        preferred_element_type=jnp.float32,
    ) + b_ref[...] * float(P)


def _matmul(s, w, b2):
    return pl.pallas_call(
        _mm_body,
        out_shape=jax.ShapeDtypeStruct((B, OUT), jnp.float32),
    )(s, w, b2)


@jax.jit
def kernel(pert_batch, W, b):
    s = _seg_sum(pert_batch)
    return _matmul(s, W, b.reshape(1, OUT))


# speedup vs baseline: 7.5573x; 1.2161x over previous
import functools

import jax
import jax.numpy as jnp
from jax import lax
from jax.experimental import pallas as pl
from jax.experimental.pallas import tpu as pltpu
from jax.experimental.pallas import tpu_sc as plsc

B, P, D, OUT = 4096, 32, 128, 128
NC, NS = 2, 16
NW = NC * NS
BPW = B // NW
BLK = 8
NBLK = BPW // BLK
LANES = 16
DC = D // LANES


def _seg_sum_body(x_hbm, out_hbm, buf0, buf1, out_stage, sem0, sem1):
    c = lax.axis_index("c")
    s = lax.axis_index("s")
    wid = s * NC + c
    base = wid * BPW
    bufs = (buf0, buf1)
    sems = (sem0, sem1)

    cps = [
        pltpu.async_copy(x_hbm.at[pl.ds(base + i * BLK, BLK)], bufs[i], sems[i])
        for i in range(2)
    ]
    for g in range(NBLK):
        slot = g % 2
        cps[slot].wait()
        buf = bufs[slot]
        for b in range(BLK):
            def pbody(p, accs):
                return tuple(accs[j] + buf[b, p, pl.ds(j * LANES, LANES)]
                             for j in range(DC))
            init = tuple(jnp.zeros((LANES,), jnp.float32) for _ in range(DC))
            accs = lax.fori_loop(0, P, pbody, init)
            for j in range(DC):
                out_stage[g * BLK + b, pl.ds(j * LANES, LANES)] = accs[j]
        nxt = g + 2
        if nxt < NBLK:
            cps[slot] = pltpu.async_copy(
                x_hbm.at[pl.ds(base + nxt * BLK, BLK)], buf, sems[slot])
    pltpu.sync_copy(out_stage, out_hbm.at[pl.ds(base, BPW)])


_seg_sum = pl.kernel(
    _seg_sum_body,
    out_type=jax.ShapeDtypeStruct((B, D), jnp.float32),
    mesh=plsc.VectorSubcoreMesh(core_axis_name="c", subcore_axis_name="s"),
    scratch_types=[
        pltpu.VMEM((BLK, P, D), jnp.float32),
        pltpu.VMEM((BLK, P, D), jnp.float32),
        pltpu.VMEM((BPW, D), jnp.float32),
        pltpu.SemaphoreType.DMA,
        pltpu.SemaphoreType.DMA,
    ],
)


def _mm_body(s_ref, w_ref, b_ref, o_ref):
    o_ref[...] = lax.dot_general(
        s_ref[...], w_ref[...],
        (((1,), (1,)), ((), ())),
        preferred_element_type=jnp.float32,
    ) + b_ref[...] * float(P)


def _matmul(s, w, b2):
    return pl.pallas_call(
        _mm_body,
        out_shape=jax.ShapeDtypeStruct((B, OUT), jnp.float32),
    )(s, w, b2)


@jax.jit
def kernel(pert_batch, W, b):
    s = _seg_sum(pert_batch)
    return _matmul(s, W, b.reshape(1, OUT))
